# baseline (speedup 1.0000x reference)
import jax
import jax.numpy as jnp
from jax.experimental import pallas as pl
from jax.experimental.pallas import tpu as pltpu

_BN = 4096


def _outer(p_ref, x_hbm, o_hbm):
    def _inner(idx, x_ref, o_ref):
        tt = idx[1]
        acc = jax.lax.dot_general(
            x_ref[...], p_ref[tt],
            dimension_numbers=(((1,), (1,)), ((), ())),
            preferred_element_type=jnp.float32,
        )
        o_ref[0] = jnp.where(acc < 0, jnp.float32(0.0), jnp.float32(1.0))

    pipe = pltpu.emit_pipeline(
        _inner,
        grid=(x_hbm.shape[0] // _BN, o_hbm.shape[0]),
        in_specs=[pl.BlockSpec((_BN, x_hbm.shape[1]), lambda i, t: (i, 0))],
        out_specs=[pl.BlockSpec((1, _BN, o_hbm.shape[2]), lambda i, t: (t, i, 0))],
        _explicit_indices=True,
    )
    pipe(x_hbm, o_hbm)


def kernel(input_points, planes):
    n, d = input_points.shape
    t, h, _ = planes.shape
    return pl.pallas_call(
        _outer,
        in_specs=[
            pl.BlockSpec((t, h, d), lambda: (0, 0, 0)),
            pl.BlockSpec(memory_space=pl.ANY),
        ],
        out_specs=pl.BlockSpec(memory_space=pl.ANY),
        out_shape=jax.ShapeDtypeStruct((t, n, h), jnp.float32),
    )(planes, input_points)


# speedup vs baseline: 1.3425x; 1.3425x over previous
import jax
import jax.numpy as jnp
from jax.experimental import pallas as pl

_BN = 4096


def _lsh_block_kernel(x_ref, p_ref, o_ref):
    x = x_ref[...]
    for t in range(o_ref.shape[0]):
        acc = jax.lax.dot_general(
            x, p_ref[t],
            dimension_numbers=(((1,), (1,)), ((), ())),
            preferred_element_type=jnp.float32,
        )
        o_ref[t] = jnp.where(acc < 0, jnp.float32(0.0), jnp.float32(1.0))


def kernel(input_points, planes):
    n, d = input_points.shape
    t, h, _ = planes.shape
    return pl.pallas_call(
        _lsh_block_kernel,
        grid=(n // _BN,),
        in_specs=[
            pl.BlockSpec((_BN, d), lambda i: (i, 0)),
            pl.BlockSpec((t, h, d), lambda i: (0, 0, 0)),
        ],
        out_specs=pl.BlockSpec((t, _BN, h), lambda i: (0, i, 0)),
        out_shape=jax.ShapeDtypeStruct((t, n, h), jnp.float32),
    )(input_points, planes)
